# Initial kernel scaffold; baseline (speedup 1.0000x reference)
import jax
import jax.numpy as jnp
from jax.experimental import pallas as pl


def kernel(x, edge_index, W1, b1, W2, b2, W3, b3):
    raise NotImplementedError("write your pallas kernel here")


# speedup vs baseline: 5.4353x; 5.4353x over previous
import functools

import jax
import jax.numpy as jnp
from jax import lax
from jax.experimental import pallas as pl
from jax.experimental.pallas import tpu as pltpu
from jax.experimental.pallas import tpu_sc as plsc

N = 10000
E = 320000
F_IN = 128
H = 128
C = 40
D3 = 48

NP = 10240
NC = 2
NS = 16
NW = NC * NS
EPT = E // NW
CH = 80
NCHUNK = EPT // CH
RPS = NP // NS
CL = 16


def _sc_mesh():
    return plsc.VectorSubcoreMesh(
        core_axis_name="c", subcore_axis_name="s", num_cores=NC, num_subcores=NS
    )


def _make_agg(d, with_cnt):
    out_type = [jax.ShapeDtypeStruct((NC, NP, d), jnp.float32)]
    scratch = [
        pltpu.VMEM_SHARED((NP, d), jnp.float32),
        pltpu.VMEM((CH,), jnp.int32),
        pltpu.VMEM((CH,), jnp.int32),
        pltpu.VMEM((CH, d), jnp.float32),
        pltpu.SemaphoreType.DMA,
    ]
    if with_cnt:
        out_type.append(jax.ShapeDtypeStruct((NC, NP, CL), jnp.float32))
        scratch += [
            pltpu.VMEM_SHARED((NP, CL), jnp.float32),
            pltpu.VMEM((CH, CL), jnp.float32),
        ]

    def body(h_hbm, row_hbm, col_hbm, z_hbm, *rest):
        if with_cnt:
            (z16_hbm, out_hbm, cnt_hbm, acc, ridx, cidx, rows, sem,
             cacc, ones) = rest
        else:
            out_hbm, acc, ridx, cidx, rows, sem = rest
        c = lax.axis_index("c")
        s = lax.axis_index("s")
        rs = pl.ds(s * RPS, RPS)
        pltpu.sync_copy(z_hbm.at[rs], acc.at[rs])
        if with_cnt:
            pltpu.sync_copy(z16_hbm.at[rs], cacc.at[rs])

            def fill_ones(i, carry):
                ones[i, pl.ds(0, CL)] = jnp.ones((CL,), jnp.float32)
                return carry

            lax.fori_loop(0, CH, fill_ones, 0)
        plsc.subcore_barrier()

        base = (c * NS + s) * EPT

        def step(i, carry):
            off = base + i * CH
            pltpu.sync_copy(row_hbm.at[pl.ds(off, CH)], ridx)
            pltpu.sync_copy(col_hbm.at[pl.ds(off, CH)], cidx)
            pltpu.async_copy(h_hbm.at[ridx], rows, sem).wait()
            pltpu.sync_copy(rows, acc.at[cidx], add=True)
            if with_cnt:
                pltpu.sync_copy(ones, cacc.at[cidx], add=True)
            return carry

        lax.fori_loop(0, NCHUNK, step, 0)
        plsc.subcore_barrier()
        pltpu.sync_copy(acc.at[rs], out_hbm.at[c, rs])
        if with_cnt:
            pltpu.sync_copy(cacc.at[rs], cnt_hbm.at[c, rs])

    return pl.kernel(
        body, out_type=out_type, mesh=_sc_mesh(), scratch_types=scratch,
        compiler_params=pltpu.CompilerParams(use_tc_tiling_on_sc=False),
    )


_agg1 = _make_agg(H, True)
_agg2 = _make_agg(H, False)
_agg3 = _make_agg(D3, False)


def _linear_body(x_ref, w_ref, b_ref, o_ref):
    o_ref[...] = (
        lax.dot_general(
            x_ref[...], w_ref[...], (((1,), (1,)), ((), ())),
            preferred_element_type=jnp.float32,
        )
        + b_ref[...]
    )


def _linear(x, w, b):
    return pl.pallas_call(
        _linear_body,
        out_shape=jax.ShapeDtypeStruct((x.shape[0], w.shape[0]), jnp.float32),
    )(x, w, b)


def _combine(s_ref, c_ref, h_ref):
    sv = s_ref[...]
    cv = c_ref[...]
    ssum = sv[0, :N] + sv[1, :N]
    cnt = jnp.max(cv[0, :N] + cv[1, :N], axis=1, keepdims=True)
    mean = ssum / jnp.maximum(cnt, 1.0)
    return jnp.where(cnt > 0.0, mean, h_ref[...])


def _combine_linear_body(s_ref, c_ref, h_ref, w_ref, b_ref, o_ref):
    g = jnp.maximum(_combine(s_ref, c_ref, h_ref), 0.0)
    o_ref[...] = (
        lax.dot_general(
            g, w_ref[...], (((1,), (1,)), ((), ())),
            preferred_element_type=jnp.float32,
        )
        + b_ref[...]
    )


def _combine_linear(s, cnt, h, w, b):
    return pl.pallas_call(
        _combine_linear_body,
        out_shape=jax.ShapeDtypeStruct((h.shape[0], w.shape[0]), jnp.float32),
    )(s, cnt, h, w, b)


def _combine_lsm_body(s_ref, c_ref, h_ref, o_ref):
    g = _combine(s_ref, c_ref, h_ref)
    lane = lax.broadcasted_iota(jnp.int32, g.shape, 1)
    valid = lane < C
    gm = jnp.where(valid, g, -jnp.inf)
    m = jnp.max(gm, axis=1, keepdims=True)
    ex = jnp.where(valid, jnp.exp(g - m), 0.0)
    lse = jnp.log(jnp.sum(ex, axis=1, keepdims=True)) + m
    o_ref[...] = (g - lse)[:, :C]


def _combine_lsm(s, cnt, h):
    return pl.pallas_call(
        _combine_lsm_body,
        out_shape=jax.ShapeDtypeStruct((h.shape[0], C), jnp.float32),
    )(s, cnt, h)


def kernel(x, edge_index, W1, b1, W2, b2, W3, b3):
    row = edge_index[0]
    col = edge_index[1]
    z128 = jnp.zeros((NP, H), jnp.float32)
    z48 = jnp.zeros((NP, D3), jnp.float32)
    z16 = jnp.zeros((NP, CL), jnp.float32)
    w3p = jnp.zeros((D3, H), jnp.float32).at[:C].set(W3)
    b3p = jnp.zeros((1, D3), jnp.float32).at[:, :C].set(b3)

    h1 = _linear(x, W1, b1.reshape(1, -1))
    s1, cnt = _agg1(h1, row, col, z128, z16)
    h2 = _combine_linear(s1, cnt, h1, W2, b2.reshape(1, -1))
    (s2,) = _agg2(h2, row, col, z128)
    h3 = _combine_linear(s2, cnt, h2, w3p, b3p)
    (s3,) = _agg3(h3, row, col, z48)
    return _combine_lsm(s3, cnt, h3)
